# baseline (speedup 1.0000x reference)
import functools

import jax
import jax.numpy as jnp
from jax import lax
from jax.experimental import pallas as pl
from jax.experimental.pallas import tpu as pltpu
from jax.experimental.pallas import tpu_sc as plsc

_NC = 2
_NS = 16
_NW = _NC * _NS

_ROWS_PER_CHUNK = 64


def _sc_add(seq, dim, n_rows, x_hbm, pos_hbm, out_hbm, xbuf, posbuf):
    wid = lax.axis_index("c") * _NS + lax.axis_index("s")
    rows_per_w = n_rows // _NW
    batches_per_w = rows_per_w // seq
    chunk_words = _ROWS_PER_CHUNK * dim
    chunks_per_seq = seq // _ROWS_PER_CHUNK
    base = wid * rows_per_w * dim

    for pc in range(chunks_per_seq):
        pltpu.sync_copy(pos_hbm.at[pl.ds(pc * chunk_words, chunk_words)], posbuf)
        for b in range(batches_per_w):
            off = base + (b * seq + pc * _ROWS_PER_CHUNK) * dim
            pltpu.sync_copy(x_hbm.at[pl.ds(off, chunk_words)], xbuf)

            @pl.loop(0, chunk_words // 16, unroll=8)
            def _(i):
                s = pl.ds(i * 16, 16)
                xbuf[s] = xbuf[s] + posbuf[s]

            pltpu.sync_copy(xbuf, out_hbm.at[pl.ds(off, chunk_words)])


def kernel(patch_embeddings, pos_table):
    batch, seq, dim = patch_embeddings.shape
    n_rows = batch * seq
    chunk_words = _ROWS_PER_CHUNK * dim
    pos = pos_table[:seq].reshape(-1)
    x = patch_embeddings.reshape(-1)

    mesh = plsc.VectorSubcoreMesh(core_axis_name="c", subcore_axis_name="s")
    out = pl.kernel(
        functools.partial(_sc_add, seq, dim, n_rows),
        out_type=jax.ShapeDtypeStruct((n_rows * dim,), patch_embeddings.dtype),
        mesh=mesh,
        scratch_types=[
            pltpu.VMEM((chunk_words,), jnp.float32),
            pltpu.VMEM((chunk_words,), jnp.float32),
        ],
    )(x, pos)
    return out.reshape(batch, seq, dim)


# speedup vs baseline: 1.4951x; 1.4951x over previous
import functools

import jax
import jax.numpy as jnp
from jax import lax
from jax.experimental import pallas as pl
from jax.experimental.pallas import tpu as pltpu
from jax.experimental.pallas import tpu_sc as plsc

_NC = 2
_NS = 16
_NBG = 4
_NSR = 8

_SUB = 4
_CHUNK_ROWS = 18
_ROWS_PER_SR = _CHUNK_ROWS * _SUB


def _sc_add(batch, seq, dim, x_hbm, pos_hbm, out_hbm,
            xb0, xb1, xb2, xb3, pb,
            si0, si1, si2, si3, so0, so1, so2, so3):
    xb = (xb0, xb1, xb2, xb3)
    si = (si0, si1, si2, si3)
    so = (so0, so1, so2, so3)
    wid = lax.axis_index("c") * _NS + lax.axis_index("s")
    bg = wid % _NBG
    sr = wid // _NBG
    bpg = batch // _NBG
    cw = _CHUNK_ROWS * dim

    pos_off = sr * _ROWS_PER_SR * dim
    pltpu.sync_copy(pos_hbm.at[pl.ds(pos_off, _SUB * cw)], pb)

    def x_slice(m, k):
        off = (bg * bpg + m) * seq * dim + pos_off + k * cw
        return pl.ds(off, cw)

    def wait_in(k):
        pltpu.make_async_copy(x_hbm.at[x_slice(0, k)], xb[k], si[k]).wait()

    def start_in(m, k):
        pltpu.async_copy(x_hbm.at[x_slice(m, k)], xb[k], si[k])

    def wait_out(k):
        pltpu.make_async_copy(xb[k], out_hbm.at[x_slice(0, k)], so[k]).wait()

    def start_out(m, k):
        pltpu.async_copy(xb[k], out_hbm.at[x_slice(m, k)], so[k])

    def compute(k):
        buf = xb[k]
        pos_base = k * cw

        @pl.loop(0, cw // 16, unroll=8)
        def _(i):
            s = pl.ds(i * 16, 16)
            buf[s] = buf[s] + pb[pl.ds(pos_base + i * 16, 16)]

    def body(m, k, do_tail):
        wait_in(k)
        compute(k)
        start_out(m, k)
        if do_tail:
            k2 = (k + 2) % _SUB
            m_prev = m if k >= 2 else m - 1
            m_next = m if k + 2 < _SUB else m + 1
            wait_out(k2)
            start_in(m_next, k2)

    for k in range(_SUB):
        start_in(0, k)

    for k in range(_SUB):
        body(0, k, do_tail=(k >= 2))

    @pl.loop(1, bpg - 1)
    def _(m):
        for k in range(_SUB):
            body(m, k, do_tail=True)

    for k in range(_SUB):
        body(bpg - 1, k, do_tail=(k < 2))

    for k in range(_SUB):
        wait_out(k)


def kernel(patch_embeddings, pos_table):
    batch, seq, dim = patch_embeddings.shape
    cw = _CHUNK_ROWS * dim
    pos = pos_table[:seq].reshape(-1)
    x = patch_embeddings.reshape(-1)

    mesh = plsc.VectorSubcoreMesh(core_axis_name="c", subcore_axis_name="s")
    out = pl.kernel(
        functools.partial(_sc_add, batch, seq, dim),
        out_type=jax.ShapeDtypeStruct((batch * seq * dim,), patch_embeddings.dtype),
        mesh=mesh,
        scratch_types=[
            pltpu.VMEM((cw,), jnp.float32),
            pltpu.VMEM((cw,), jnp.float32),
            pltpu.VMEM((cw,), jnp.float32),
            pltpu.VMEM((cw,), jnp.float32),
            pltpu.VMEM((_SUB * cw,), jnp.float32),
            pltpu.SemaphoreType.DMA,
            pltpu.SemaphoreType.DMA,
            pltpu.SemaphoreType.DMA,
            pltpu.SemaphoreType.DMA,
            pltpu.SemaphoreType.DMA,
            pltpu.SemaphoreType.DMA,
            pltpu.SemaphoreType.DMA,
            pltpu.SemaphoreType.DMA,
        ],
    )(x, pos)
    return out.reshape(batch, seq, dim)


# speedup vs baseline: 1.8854x; 1.2610x over previous
import functools

import jax
import jax.numpy as jnp
from jax import lax
from jax.experimental import pallas as pl
from jax.experimental.pallas import tpu as pltpu
from jax.experimental.pallas import tpu_sc as plsc

_NC = 2
_NS = 16
_NBG = 4
_NSR = 8

_SUB = 4
_CHUNK_ROWS = 18
_ROWS_PER_SR = _CHUNK_ROWS * _SUB


def _sc_add(batch, seq, dim, x_hbm, pos_hbm, out_hbm,
            xb0, xb1, xb2, xb3, pb,
            si0, si1, si2, si3, so0, so1, so2, so3):
    xb = (xb0, xb1, xb2, xb3)
    si = (si0, si1, si2, si3)
    so = (so0, so1, so2, so3)
    wid = lax.axis_index("c") * _NS + lax.axis_index("s")
    bg = wid % _NBG
    sr = wid // _NBG
    bpg = batch // _NBG
    cw = _CHUNK_ROWS * dim

    pos_off = sr * _ROWS_PER_SR * dim
    pltpu.sync_copy(pos_hbm.at[pl.ds(pos_off, _SUB * cw)], pb)

    def x_slice(m, k):
        off = (bg * bpg + m) * seq * dim + pos_off + k * cw
        return pl.ds(off, cw)

    def wait_in(k):
        pltpu.make_async_copy(x_hbm.at[x_slice(0, k)], xb[k], si[k]).wait()

    def start_in(m, k):
        pltpu.async_copy(x_hbm.at[x_slice(m, k)], xb[k], si[k])

    def wait_out(k):
        pltpu.make_async_copy(xb[k], out_hbm.at[x_slice(0, k)], so[k]).wait()

    def start_out(m, k):
        pltpu.async_copy(xb[k], out_hbm.at[x_slice(m, k)], so[k])

    def compute(k):
        buf = xb[k]
        pos_base = k * cw

        @plsc.parallel_loop(0, cw, step=16, unroll=8)
        def _(i):
            s = pl.ds(i, 16)
            buf[s] = buf[s] + pb[pl.ds(pos_base + i, 16)]

    def body(m, k, do_tail):
        wait_in(k)
        compute(k)
        start_out(m, k)
        if do_tail:
            k2 = (k + 2) % _SUB
            m_prev = m if k >= 2 else m - 1
            m_next = m if k + 2 < _SUB else m + 1
            wait_out(k2)
            start_in(m_next, k2)

    for k in range(_SUB):
        start_in(0, k)

    for k in range(_SUB):
        body(0, k, do_tail=(k >= 2))

    @pl.loop(1, bpg - 1)
    def _(m):
        for k in range(_SUB):
            body(m, k, do_tail=True)

    for k in range(_SUB):
        body(bpg - 1, k, do_tail=(k < 2))

    for k in range(_SUB):
        wait_out(k)


def kernel(patch_embeddings, pos_table):
    batch, seq, dim = patch_embeddings.shape
    cw = _CHUNK_ROWS * dim
    pos = pos_table[:seq].reshape(-1)
    x = patch_embeddings.reshape(-1)

    mesh = plsc.VectorSubcoreMesh(core_axis_name="c", subcore_axis_name="s")
    out = pl.kernel(
        functools.partial(_sc_add, batch, seq, dim),
        out_type=jax.ShapeDtypeStruct((batch * seq * dim,), patch_embeddings.dtype),
        mesh=mesh,
        scratch_types=[
            pltpu.VMEM((cw,), jnp.float32),
            pltpu.VMEM((cw,), jnp.float32),
            pltpu.VMEM((cw,), jnp.float32),
            pltpu.VMEM((cw,), jnp.float32),
            pltpu.VMEM((_SUB * cw,), jnp.float32),
            pltpu.SemaphoreType.DMA,
            pltpu.SemaphoreType.DMA,
            pltpu.SemaphoreType.DMA,
            pltpu.SemaphoreType.DMA,
            pltpu.SemaphoreType.DMA,
            pltpu.SemaphoreType.DMA,
            pltpu.SemaphoreType.DMA,
            pltpu.SemaphoreType.DMA,
        ],
    )(x, pos)
    return out.reshape(batch, seq, dim)


# speedup vs baseline: 5.5470x; 2.9421x over previous
import functools

import jax
import jax.numpy as jnp
from jax import lax
from jax.experimental import pallas as pl
from jax.experimental.pallas import tpu as pltpu
from jax.experimental.pallas import tpu_sc as plsc

_NC = 2
_NS = 16
_NBG = 4
_NSR = 8

_CPS = 3
_NSLOT = 4
_CHUNK_ROWS = 24
_ROWS_PER_SR = _CHUNK_ROWS * _CPS
_BLK = _CPS * _NSLOT


def _sc_add(batch, seq, dim, x_hbm, pos_hbm, out_hbm,
            xb0, xb1, xb2, xb3, pb,
            si0, si1, si2, si3, so0, so1, so2, so3):
    xb = (xb0, xb1, xb2, xb3)
    si = (si0, si1, si2, si3)
    so = (so0, so1, so2, so3)
    wid = lax.axis_index("c") * _NS + lax.axis_index("s")
    bg = wid % _NBG
    sr = wid // _NBG
    bpg = batch // _NBG
    cw = _CHUNK_ROWS * dim
    nb = bpg * _CPS
    r0 = sr * _ROWS_PER_SR

    pltpu.sync_copy(pos_hbm.at[pl.ds(r0, _ROWS_PER_SR), :], pb)

    def x_at(ref, m, c):
        return ref.at[bg * bpg + m, pl.ds(r0 + c * _CHUNK_ROWS, _CHUNK_ROWS), :]

    def wait_in(k):
        pltpu.make_async_copy(x_at(x_hbm, 0, 0), xb[k], si[k]).wait()

    def start_in(m, c, k):
        pltpu.async_copy(x_at(x_hbm, m, c), xb[k], si[k])

    def wait_out(k):
        pltpu.make_async_copy(xb[k], x_at(out_hbm, 0, 0), so[k]).wait()

    def start_out(m, c, k):
        pltpu.async_copy(xb[k], x_at(out_hbm, m, c), so[k])

    def compute(c, k):
        buf = xb[k]

        @pl.loop(0, _CHUNK_ROWS)
        def _(r):
            @plsc.parallel_loop(0, dim, step=16, unroll=8)
            def _(j):
                s = pl.ds(j, 16)
                buf[r, s] = buf[r, s] + pb[c * _CHUNK_ROWS + r, s]

    def body(blk, u, tail, skip_wait=False):
        m, c, k = u // _CPS, u % _CPS, u % _NSLOT
        m = _NSLOT * blk + m
        wait_in(k)
        compute(c, k)
        start_out(m, c, k)
        if tail:
            u2 = u + 2
            m2, c2, k2 = u2 // _CPS, u2 % _CPS, u2 % _NSLOT
            if u2 >= _BLK:
                m2, c2, k2 = (u2 - _BLK) // _CPS, (u2 - _BLK) % _CPS, (u2 - _BLK) % _NSLOT
                m2 += _NSLOT
            m2 = _NSLOT * blk + m2
            if not skip_wait:
                wait_out(k2)
            start_in(m2, c2, k2)

    n_blk = nb // _BLK

    start_in(0, 0, 0)
    start_in(0, 1, 1)

    for u in range(_BLK):
        body(0, u, tail=True, skip_wait=(u < 2))

    @pl.loop(1, n_blk - 1)
    def _(blk):
        for u in range(_BLK):
            body(blk, u, tail=True)

    for u in range(_BLK):
        body(n_blk - 1, u, tail=(u < _BLK - 2))

    for k in range(_NSLOT):
        wait_out(k)


def kernel(patch_embeddings, pos_table):
    batch, seq, dim = patch_embeddings.shape
    pos = pos_table[:seq]

    mesh = plsc.VectorSubcoreMesh(core_axis_name="c", subcore_axis_name="s")
    out = pl.kernel(
        functools.partial(_sc_add, batch, seq, dim),
        out_type=jax.ShapeDtypeStruct((batch, seq, dim), patch_embeddings.dtype),
        mesh=mesh,
        scratch_types=[
            pltpu.VMEM((_CHUNK_ROWS, dim), jnp.float32),
            pltpu.VMEM((_CHUNK_ROWS, dim), jnp.float32),
            pltpu.VMEM((_CHUNK_ROWS, dim), jnp.float32),
            pltpu.VMEM((_CHUNK_ROWS, dim), jnp.float32),
            pltpu.VMEM((_ROWS_PER_SR, dim), jnp.float32),
            pltpu.SemaphoreType.DMA,
            pltpu.SemaphoreType.DMA,
            pltpu.SemaphoreType.DMA,
            pltpu.SemaphoreType.DMA,
            pltpu.SemaphoreType.DMA,
            pltpu.SemaphoreType.DMA,
            pltpu.SemaphoreType.DMA,
            pltpu.SemaphoreType.DMA,
        ],
    )(patch_embeddings, pos)
    return out
